# baseline (speedup 1.0000x reference)
import functools

import jax
import jax.numpy as jnp
from jax import lax
from jax.experimental import pallas as pl
from jax.experimental.pallas import tpu as pltpu
from jax.experimental.pallas import tpu_sc as plsc

DIM = 64
NC = 2
NS = 16
NW = NC * NS

GATHER_CHUNK = 1024
TR_COLS = 8192
FMT_ROWS = 512


def _tc_transform(tableT, W_down, W_up, vocab):
    def tr_kernel(tt_ref, wd_ref, wu_ref, out_ref):
        wc = jnp.dot(wu_ref[...], wd_ref[...],
                     preferred_element_type=jnp.float32)
        z = lax.dot_general(tt_ref[...], wc, (((0,), (1,)), ((), ())),
                            preferred_element_type=jnp.float32)
        out_ref[0, :, 0:DIM] = z[: TR_COLS // 2]
        out_ref[0, :, DIM:128] = z[TR_COLS // 2 :]

    grid = (vocab + TR_COLS - 1) // TR_COLS
    return pl.pallas_call(
        tr_kernel,
        grid=(grid,),
        in_specs=[
            pl.BlockSpec((DIM, TR_COLS), lambda i: (0, i)),
            pl.BlockSpec((DIM // 2, DIM), lambda i: (0, 0)),
            pl.BlockSpec((DIM, DIM // 2), lambda i: (0, 0)),
        ],
        out_specs=pl.BlockSpec((1, TR_COLS // 2, 128), lambda i: (i, 0, 0)),
        out_shape=jax.ShapeDtypeStruct((grid, TR_COLS // 2, 128), jnp.float32),
        compiler_params=pltpu.CompilerParams(fuse_transposed_lhs_in_matmul=True),
    )(tableT, W_down, W_up)


def _sc_gather(src, flat_idx, n):
    per_w = n // NW
    chunk = min(GATHER_CHUNK, per_w)
    n_chunks = per_w // chunk
    mesh = plsc.VectorSubcoreMesh(
        core_axis_name="c", subcore_axis_name="s",
        num_cores=NC, num_subcores=NS)

    @functools.partial(
        pl.kernel,
        out_type=jax.ShapeDtypeStruct((n, DIM), jnp.float32),
        mesh=mesh,
        scratch_types=[
            pltpu.VMEM((chunk,), jnp.int32),
            pltpu.VMEM((chunk, DIM), jnp.float32),
            pltpu.SemaphoreType.DMA,
        ],
        compiler_params=pltpu.CompilerParams(use_tc_tiling_on_sc=False),
    )
    def gather_kernel(src_hbm, idx_hbm, out_hbm, idx_v, rows_v, sem):
        wid = lax.axis_index("s") * NC + lax.axis_index("c")
        base = wid * per_w

        def body(i, carry):
            start = base + i * chunk
            pltpu.sync_copy(idx_hbm.at[pl.ds(start, chunk)], idx_v)
            pltpu.async_copy(src_hbm.at[idx_v], rows_v, sem).wait()
            pltpu.sync_copy(rows_v, out_hbm.at[pl.ds(start, chunk)])
            return carry

        lax.fori_loop(0, n_chunks, body, 0)

    return gather_kernel(src, flat_idx)


def _tc_format(e2, B, L):
    sub = FMT_ROWS // 128

    def fmt_kernel(e_ref, out_ref):
        for s in range(sub):
            rows = e_ref[s * 128 : (s + 1) * 128, :]
            left = jnp.transpose(rows[:, 0:DIM])
            right = jnp.transpose(rows[:, DIM:128])
            out_ref[0, :, 2 * s, :, :] = left.reshape(8, 8, 128)
            out_ref[0, :, 2 * s + 1, :, :] = right.reshape(8, 8, 128)

    n_bg = B // 128
    bg_per_step = 2 * sub
    grid = (L, n_bg // bg_per_step)
    return pl.pallas_call(
        fmt_kernel,
        grid=grid,
        in_specs=[
            pl.BlockSpec((FMT_ROWS, 128),
                         lambda l, k: (l * (n_bg // bg_per_step) + k, 0)),
        ],
        out_specs=pl.BlockSpec((1, 8, bg_per_step, 8, 128),
                               lambda l, k: (l, 0, k, 0, 0)),
        out_shape=jax.ShapeDtypeStruct((L, 8, n_bg, 8, 128), jnp.float32),
    )(e2)


@jax.jit
def kernel(x, table, W_down, W_up):
    B, L = x.shape
    vocab = table.shape[0]
    n = B * L

    idx = x.T.reshape(L, B // 256, 2, 128).swapaxes(2, 3).reshape(n)
    idx = idx.astype(jnp.int32)

    blk = idx >> 13
    off = idx & (TR_COLS - 1)
    phys_idx = (blk << 13) + ((off & (TR_COLS // 2 - 1)) << 1) \
        + (off >> 12)

    g = _tc_transform(table.T, W_down, W_up, vocab)
    g_rows = g.reshape(g.shape[0] * g.shape[1] * 2, DIM)
    e = _sc_gather(g_rows, phys_idx, n)
    e2 = e.reshape(n // 2, 128)
    y = _tc_format(e2, B, L)
    return y.transpose(2, 4, 0, 1, 3).reshape(B, L, DIM)


# speedup vs baseline: 1.5928x; 1.5928x over previous
import functools

import jax
import jax.numpy as jnp
from jax import lax
from jax.experimental import pallas as pl
from jax.experimental.pallas import tpu as pltpu
from jax.experimental.pallas import tpu_sc as plsc

DIM = 64
NC = 2
NS = 16
NW = NC * NS
LANES = 16

GATHER_CHUNK = 1024
TR_COLS = 8192
FMT_ROWS = 1024


def _tc_transform(tableT, W_down, W_up, vocab):
    def tr_kernel(tt_ref, wd_ref, wu_ref, out_ref):
        wc = jnp.dot(wu_ref[...], wd_ref[...],
                     preferred_element_type=jnp.float32)
        z = lax.dot_general(tt_ref[...], wc, (((0,), (1,)), ((), ())),
                            preferred_element_type=jnp.float32)
        out_ref[0, :, 0:DIM] = z[: TR_COLS // 2]
        out_ref[0, :, DIM:128] = z[TR_COLS // 2 :]

    grid = (vocab + TR_COLS - 1) // TR_COLS
    return pl.pallas_call(
        tr_kernel,
        grid=(grid,),
        in_specs=[
            pl.BlockSpec((DIM, TR_COLS), lambda i: (0, i)),
            pl.BlockSpec((DIM // 2, DIM), lambda i: (0, 0)),
            pl.BlockSpec((DIM, DIM // 2), lambda i: (0, 0)),
        ],
        out_specs=pl.BlockSpec((1, TR_COLS // 2, 128), lambda i: (i, 0, 0)),
        out_shape=jax.ShapeDtypeStruct((grid, TR_COLS // 2, 128), jnp.float32),
        compiler_params=pltpu.CompilerParams(fuse_transposed_lhs_in_matmul=True),
    )(tableT, W_down, W_up)


def _sc_gather(src, flat_idx, n, B):
    half = B // 2
    pairs = n // 2
    per_w = pairs // NW
    chunk = min(GATHER_CHUNK // 2, per_w)
    n_chunks = per_w // chunk
    mesh = plsc.VectorSubcoreMesh(
        core_axis_name="c", subcore_axis_name="s",
        num_cores=NC, num_subcores=NS)

    @functools.partial(
        pl.kernel,
        out_type=jax.ShapeDtypeStruct((pairs, 2 * DIM), jnp.float32),
        mesh=mesh,
        scratch_types=[
            pltpu.VMEM((chunk,), jnp.int32),
            pltpu.VMEM((chunk,), jnp.int32),
            pltpu.VMEM((chunk, DIM), jnp.float32),
            pltpu.VMEM((chunk, DIM), jnp.float32),
            pltpu.SemaphoreType.DMA,
            pltpu.SemaphoreType.DMA,
        ],
        compiler_params=pltpu.CompilerParams(use_tc_tiling_on_sc=False),
    )
    def gather_kernel(src_hbm, idx_hbm, out_hbm,
                      idx_a, idx_b, rows_a, rows_b, sem_a, sem_b):
        wid = lax.axis_index("s") * NC + lax.axis_index("c")
        base = wid * per_w

        def body(i, carry):
            m0 = pl.multiple_of(base + i * chunk, chunk)
            l = m0 >> 13
            u0 = m0 - (l << 13)
            pos = pl.multiple_of((l << 14) + u0, chunk)
            pltpu.sync_copy(idx_hbm.at[pl.ds(pos, chunk)], idx_a)
            pltpu.sync_copy(idx_hbm.at[pl.ds(pos + half, chunk)], idx_b)
            ca = pltpu.async_copy(src_hbm.at[idx_a], rows_a, sem_a)
            cb = pltpu.async_copy(src_hbm.at[idx_b], rows_b, sem_b)
            ca.wait()
            pltpu.sync_copy(rows_a, out_hbm.at[pl.ds(m0, chunk), pl.ds(0, DIM)])
            cb.wait()
            pltpu.sync_copy(rows_b, out_hbm.at[pl.ds(m0, chunk), pl.ds(DIM, DIM)])
            return carry

        lax.fori_loop(0, n_chunks, body, 0)

    return gather_kernel(src, flat_idx)


def _tc_format(e2, B, L):
    sub = FMT_ROWS // 128

    def fmt_kernel(e_ref, out_ref):
        z = jnp.transpose(e_ref[...])
        for s in range(sub):
            zs = z[:, s * 128 : (s + 1) * 128]
            out_ref[0, :, 0, s, :, :] = zs[0:DIM].reshape(8, 8, 128)
            out_ref[0, :, 1, s, :, :] = zs[DIM:128].reshape(8, 8, 128)

    n_bgh = B // 256
    grid = (L, n_bgh // sub)
    return pl.pallas_call(
        fmt_kernel,
        grid=grid,
        in_specs=[
            pl.BlockSpec((FMT_ROWS, 128),
                         lambda l, k: (l * (n_bgh // sub) + k, 0)),
        ],
        out_specs=pl.BlockSpec((1, 8, 2, sub, 8, 128),
                               lambda l, k: (l, 0, 0, k, 0, 0)),
        out_shape=jax.ShapeDtypeStruct((L, 8, 2, n_bgh, 8, 128), jnp.float32),
    )(e2)


@jax.jit
def kernel(x, table, W_down, W_up):
    B, L = x.shape
    vocab = table.shape[0]
    n = B * L

    idx = x.T.reshape(n).astype(jnp.int32)

    blk = idx >> 13
    off = idx & (TR_COLS - 1)
    phys_idx = (blk << 13) + ((off & (TR_COLS // 2 - 1)) << 1) \
        + (off >> 12)

    g = _tc_transform(table.T, W_down, W_up, vocab)
    g_rows = g.reshape(g.shape[0] * g.shape[1] * 2, DIM)
    e2 = _sc_gather(g_rows, phys_idx, n, B)
    y = _tc_format(e2, B, L)
    return y.transpose(2, 3, 5, 0, 1, 4).reshape(B, L, DIM)


# speedup vs baseline: 1.8932x; 1.1886x over previous
import functools

import jax
import jax.numpy as jnp
from jax import lax
from jax.experimental import pallas as pl
from jax.experimental.pallas import tpu as pltpu
from jax.experimental.pallas import tpu_sc as plsc

DIM = 64
NC = 2
NS = 16
NW = NC * NS
LANES = 16

GATHER_CHUNK = 1024
TR_COLS = 16384
FMT_ROWS = 2048


def _tc_transform(tableT, W_down, W_up, vocab):
    def tr_kernel(tt_ref, wd_ref, wu_ref, out_ref):
        wc = jnp.dot(wu_ref[...], wd_ref[...],
                     preferred_element_type=jnp.float32)
        z = lax.dot_general(tt_ref[...], wc, (((0,), (1,)), ((), ())),
                            preferred_element_type=jnp.float32)
        out_ref[0, :, 0:DIM] = z[: TR_COLS // 2]
        out_ref[0, :, DIM:128] = z[TR_COLS // 2 :]

    grid = (vocab + TR_COLS - 1) // TR_COLS
    return pl.pallas_call(
        tr_kernel,
        grid=(grid,),
        in_specs=[
            pl.BlockSpec((DIM, TR_COLS), lambda i: (0, i)),
            pl.BlockSpec((DIM // 2, DIM), lambda i: (0, 0)),
            pl.BlockSpec((DIM, DIM // 2), lambda i: (0, 0)),
        ],
        out_specs=pl.BlockSpec((1, TR_COLS // 2, 128), lambda i: (i, 0, 0)),
        out_shape=jax.ShapeDtypeStruct((grid, TR_COLS // 2, 128), jnp.float32),
        compiler_params=pltpu.CompilerParams(fuse_transposed_lhs_in_matmul=True),
    )(tableT, W_down, W_up)


def _sc_gather(src, flat_idx, n, B):
    half = B // 2
    pairs = n // 2
    per_w = pairs // NW
    chunk = min(GATHER_CHUNK // 2, per_w)
    n_chunks = per_w // chunk
    mesh = plsc.VectorSubcoreMesh(
        core_axis_name="c", subcore_axis_name="s",
        num_cores=NC, num_subcores=NS)

    @functools.partial(
        pl.kernel,
        out_type=jax.ShapeDtypeStruct((pairs, 2 * DIM), jnp.float32),
        mesh=mesh,
        scratch_types=[
            pltpu.VMEM((chunk,), jnp.int32),
            pltpu.VMEM((chunk,), jnp.int32),
            pltpu.VMEM((chunk, DIM), jnp.float32),
            pltpu.VMEM((chunk, DIM), jnp.float32),
            pltpu.SemaphoreType.DMA,
            pltpu.SemaphoreType.DMA,
        ],
        compiler_params=pltpu.CompilerParams(use_tc_tiling_on_sc=False),
    )
    def gather_kernel(src_hbm, idx_hbm, out_hbm,
                      idx_a, idx_b, rows_a, rows_b, sem_a, sem_b):
        wid = lax.axis_index("s") * NC + lax.axis_index("c")
        base = wid * per_w

        def body(i, carry):
            m0 = pl.multiple_of(base + i * chunk, chunk)
            l = m0 >> 13
            u0 = m0 - (l << 13)
            pos = pl.multiple_of((l << 14) + u0, chunk)
            pltpu.sync_copy(idx_hbm.at[pl.ds(pos, chunk)], idx_a)
            pltpu.sync_copy(idx_hbm.at[pl.ds(pos + half, chunk)], idx_b)
            ca = pltpu.async_copy(src_hbm.at[idx_a], rows_a, sem_a)
            cb = pltpu.async_copy(src_hbm.at[idx_b], rows_b, sem_b)
            ca.wait()
            pltpu.sync_copy(rows_a, out_hbm.at[pl.ds(m0, chunk), pl.ds(0, DIM)])
            cb.wait()
            pltpu.sync_copy(rows_b, out_hbm.at[pl.ds(m0, chunk), pl.ds(DIM, DIM)])
            return carry

        lax.fori_loop(0, n_chunks, body, 0)

    return gather_kernel(src, flat_idx)


def _tc_format(e2, B, L):
    sub = FMT_ROWS // 128

    def fmt_kernel(e_ref, out_ref):
        z = jnp.transpose(e_ref[...])
        for s in range(sub):
            zs = z[:, s * 128 : (s + 1) * 128]
            out_ref[0, :, 0, s, :, :] = zs[0:DIM].reshape(8, 8, 128)
            out_ref[0, :, 1, s, :, :] = zs[DIM:128].reshape(8, 8, 128)

    n_bgh = B // 256
    grid = (L, n_bgh // sub)
    return pl.pallas_call(
        fmt_kernel,
        grid=grid,
        in_specs=[
            pl.BlockSpec((FMT_ROWS, 128),
                         lambda l, k: (l * (n_bgh // sub) + k, 0)),
        ],
        out_specs=pl.BlockSpec((1, 8, 2, sub, 8, 128),
                               lambda l, k: (l, 0, 0, k, 0, 0)),
        out_shape=jax.ShapeDtypeStruct((L, 8, 2, n_bgh, 8, 128), jnp.float32),
    )(e2)


@jax.jit
def kernel(x, table, W_down, W_up):
    B, L = x.shape
    vocab = table.shape[0]
    n = B * L

    idx = x.T.reshape(n).astype(jnp.int32)

    tr_sh = TR_COLS.bit_length() - 1
    blk = idx >> tr_sh
    off = idx & (TR_COLS - 1)
    phys_idx = (blk << tr_sh) + ((off & (TR_COLS // 2 - 1)) << 1) \
        + (off >> (tr_sh - 1))

    g = _tc_transform(table.T, W_down, W_up, vocab)
    g_rows = g.reshape(g.shape[0] * g.shape[1] * 2, DIM)
    e2 = _sc_gather(g_rows, phys_idx, n, B)
    y = _tc_format(e2, B, L)
    return y.transpose(2, 3, 5, 0, 1, 4).reshape(B, L, DIM)


# speedup vs baseline: 2.0969x; 1.1076x over previous
import functools

import jax
import jax.numpy as jnp
from jax import lax
from jax.experimental import pallas as pl
from jax.experimental.pallas import tpu as pltpu
from jax.experimental.pallas import tpu_sc as plsc

DIM = 64
NC = 2
NS = 16
NW = NC * NS
LANES = 16

GATHER_CHUNK = 1024
TR_COLS = 16384
FMT_ROWS = 4096


def _tc_transform(tableT, W_down, W_up, vocab):
    def tr_kernel(tt_ref, wd_ref, wu_ref, out_ref):
        wc = jnp.dot(wu_ref[...], wd_ref[...],
                     preferred_element_type=jnp.float32)
        z = lax.dot_general(tt_ref[...], wc, (((0,), (1,)), ((), ())),
                            preferred_element_type=jnp.float32)
        out_ref[0, :, 0:DIM] = z[: TR_COLS // 2]
        out_ref[0, :, DIM:128] = z[TR_COLS // 2 :]

    grid = (vocab + TR_COLS - 1) // TR_COLS
    return pl.pallas_call(
        tr_kernel,
        grid=(grid,),
        in_specs=[
            pl.BlockSpec((DIM, TR_COLS), lambda i: (0, i)),
            pl.BlockSpec((DIM // 2, DIM), lambda i: (0, 0)),
            pl.BlockSpec((DIM, DIM // 2), lambda i: (0, 0)),
        ],
        out_specs=pl.BlockSpec((1, TR_COLS // 2, 128), lambda i: (i, 0, 0)),
        out_shape=jax.ShapeDtypeStruct((grid, TR_COLS // 2, 128), jnp.float32),
        compiler_params=pltpu.CompilerParams(fuse_transposed_lhs_in_matmul=True),
    )(tableT, W_down, W_up)


def _sc_gather(src, flat_idx, n, B):
    half = B // 2
    pairs = n // 2
    per_w = pairs // NW
    chunk = min(GATHER_CHUNK // 2, per_w)
    n_chunks = per_w // chunk
    mesh = plsc.VectorSubcoreMesh(
        core_axis_name="c", subcore_axis_name="s",
        num_cores=NC, num_subcores=NS)

    @functools.partial(
        pl.kernel,
        out_type=jax.ShapeDtypeStruct((pairs, 2 * DIM), jnp.float32),
        mesh=mesh,
        scratch_types=[
            pltpu.VMEM((chunk,), jnp.int32),
            pltpu.VMEM((chunk,), jnp.int32),
            pltpu.VMEM((chunk, DIM), jnp.float32),
            pltpu.VMEM((chunk, DIM), jnp.float32),
            pltpu.SemaphoreType.DMA,
            pltpu.SemaphoreType.DMA,
        ],
        compiler_params=pltpu.CompilerParams(use_tc_tiling_on_sc=False),
    )
    def gather_kernel(src_hbm, idx_hbm, out_hbm,
                      idx_a, idx_b, rows_a, rows_b, sem_a, sem_b):
        wid = lax.axis_index("s") * NC + lax.axis_index("c")
        base = wid * per_w

        def body(i, carry):
            m0 = pl.multiple_of(base + i * chunk, chunk)
            l = m0 >> 13
            u0 = m0 - (l << 13)
            pos = pl.multiple_of((l << 14) + u0, chunk)
            pltpu.sync_copy(idx_hbm.at[pl.ds(pos, chunk)], idx_a)
            pltpu.sync_copy(idx_hbm.at[pl.ds(pos + half, chunk)], idx_b)
            ca = pltpu.async_copy(src_hbm.at[idx_a], rows_a, sem_a)
            cb = pltpu.async_copy(src_hbm.at[idx_b], rows_b, sem_b)
            ca.wait()
            pltpu.sync_copy(rows_a, out_hbm.at[pl.ds(m0, chunk), pl.ds(0, DIM)])
            cb.wait()
            pltpu.sync_copy(rows_b, out_hbm.at[pl.ds(m0, chunk), pl.ds(DIM, DIM)])
            return carry

        lax.fori_loop(0, n_chunks, body, 0)

    return gather_kernel(src, flat_idx)


def _tc_format(e2, B, L):
    sub = FMT_ROWS // 128

    def fmt_kernel(e_ref, out_ref):
        z = jnp.transpose(e_ref[...])
        for s in range(sub):
            zs = z[:, s * 128 : (s + 1) * 128]
            out_ref[0, :, 0, s, :, :] = zs[0:DIM].reshape(8, 8, 128)
            out_ref[0, :, 1, s, :, :] = zs[DIM:128].reshape(8, 8, 128)

    n_bgh = B // 256
    grid = (L, n_bgh // sub)
    return pl.pallas_call(
        fmt_kernel,
        grid=grid,
        in_specs=[
            pl.BlockSpec((FMT_ROWS, 128),
                         lambda l, k: (l * (n_bgh // sub) + k, 0)),
        ],
        out_specs=pl.BlockSpec((1, 8, 2, sub, 8, 128),
                               lambda l, k: (l, 0, 0, k, 0, 0)),
        out_shape=jax.ShapeDtypeStruct((L, 8, 2, n_bgh, 8, 128), jnp.float32),
    )(e2)


@jax.jit
def kernel(x, table, W_down, W_up):
    B, L = x.shape
    vocab = table.shape[0]
    n = B * L

    idx = x.T.reshape(n).astype(jnp.int32)

    tr_sh = TR_COLS.bit_length() - 1
    blk = idx >> tr_sh
    off = idx & (TR_COLS - 1)
    phys_idx = (blk << tr_sh) + ((off & (TR_COLS // 2 - 1)) << 1) \
        + (off >> (tr_sh - 1))

    g = _tc_transform(table.T, W_down, W_up, vocab)
    g_rows = g.reshape(g.shape[0] * g.shape[1] * 2, DIM)
    e2 = _sc_gather(g_rows, phys_idx, n, B)
    y = _tc_format(e2, B, L)
    return y.transpose(2, 3, 5, 0, 1, 4).reshape(B, L, DIM)


# speedup vs baseline: 2.1913x; 1.0450x over previous
import functools

import jax
import jax.numpy as jnp
from jax import lax
from jax.experimental import pallas as pl
from jax.experimental.pallas import tpu as pltpu
from jax.experimental.pallas import tpu_sc as plsc

DIM = 64
NC = 2
NS = 16
NW = NC * NS
LANES = 16

GATHER_CHUNK = 1024
TR_COLS = 16384
FMT_ROWS = 8192


def _tc_transform(tableT, W_down, W_up, vocab):
    def tr_kernel(tt_ref, wd_ref, wu_ref, out_ref):
        wc = jnp.dot(wu_ref[...], wd_ref[...],
                     preferred_element_type=jnp.float32)
        z = lax.dot_general(tt_ref[...], wc, (((0,), (1,)), ((), ())),
                            preferred_element_type=jnp.float32)
        out_ref[0, :, 0:DIM] = z[: TR_COLS // 2]
        out_ref[0, :, DIM:128] = z[TR_COLS // 2 :]

    grid = (vocab + TR_COLS - 1) // TR_COLS
    return pl.pallas_call(
        tr_kernel,
        grid=(grid,),
        in_specs=[
            pl.BlockSpec((DIM, TR_COLS), lambda i: (0, i)),
            pl.BlockSpec((DIM // 2, DIM), lambda i: (0, 0)),
            pl.BlockSpec((DIM, DIM // 2), lambda i: (0, 0)),
        ],
        out_specs=pl.BlockSpec((1, TR_COLS // 2, 128), lambda i: (i, 0, 0)),
        out_shape=jax.ShapeDtypeStruct((grid, TR_COLS // 2, 128), jnp.float32),
        compiler_params=pltpu.CompilerParams(fuse_transposed_lhs_in_matmul=True),
    )(tableT, W_down, W_up)


def _sc_gather(src, flat_idx, n, B):
    half = B // 2
    pairs = n // 2
    per_w = pairs // NW
    chunk = min(GATHER_CHUNK // 2, per_w)
    n_chunks = per_w // chunk
    mesh = plsc.VectorSubcoreMesh(
        core_axis_name="c", subcore_axis_name="s",
        num_cores=NC, num_subcores=NS)

    @functools.partial(
        pl.kernel,
        out_type=jax.ShapeDtypeStruct((pairs, 2 * DIM), jnp.float32),
        mesh=mesh,
        scratch_types=[
            pltpu.VMEM((chunk,), jnp.int32),
            pltpu.VMEM((chunk,), jnp.int32),
            pltpu.VMEM((chunk, DIM), jnp.float32),
            pltpu.VMEM((chunk, DIM), jnp.float32),
            pltpu.SemaphoreType.DMA,
            pltpu.SemaphoreType.DMA,
        ],
        compiler_params=pltpu.CompilerParams(use_tc_tiling_on_sc=False),
    )
    def gather_kernel(src_hbm, idx_hbm, out_hbm,
                      idx_a, idx_b, rows_a, rows_b, sem_a, sem_b):
        wid = lax.axis_index("s") * NC + lax.axis_index("c")
        base = wid * per_w

        def body(i, carry):
            m0 = pl.multiple_of(base + i * chunk, chunk)
            l = m0 >> 13
            u0 = m0 - (l << 13)
            pos = pl.multiple_of((l << 14) + u0, chunk)
            pltpu.sync_copy(idx_hbm.at[pl.ds(pos, chunk)], idx_a)
            pltpu.sync_copy(idx_hbm.at[pl.ds(pos + half, chunk)], idx_b)
            ca = pltpu.async_copy(src_hbm.at[idx_a], rows_a, sem_a)
            cb = pltpu.async_copy(src_hbm.at[idx_b], rows_b, sem_b)
            ca.wait()
            pltpu.sync_copy(rows_a, out_hbm.at[pl.ds(m0, chunk), pl.ds(0, DIM)])
            cb.wait()
            pltpu.sync_copy(rows_b, out_hbm.at[pl.ds(m0, chunk), pl.ds(DIM, DIM)])
            return carry

        lax.fori_loop(0, n_chunks, body, 0)

    return gather_kernel(src, flat_idx)


def _tc_format(e2, B, L):
    sub = FMT_ROWS // 128

    def fmt_kernel(e_ref, out_ref):
        z = jnp.transpose(e_ref[...])
        for s in range(sub):
            zs = z[:, s * 128 : (s + 1) * 128]
            out_ref[0, :, 0, s, :, :] = zs[0:DIM].reshape(8, 8, 128)
            out_ref[0, :, 1, s, :, :] = zs[DIM:128].reshape(8, 8, 128)

    n_bgh = B // 256
    grid = (L, n_bgh // sub)
    return pl.pallas_call(
        fmt_kernel,
        grid=grid,
        in_specs=[
            pl.BlockSpec((FMT_ROWS, 128),
                         lambda l, k: (l * (n_bgh // sub) + k, 0)),
        ],
        out_specs=pl.BlockSpec((1, 8, 2, sub, 8, 128),
                               lambda l, k: (l, 0, 0, k, 0, 0)),
        out_shape=jax.ShapeDtypeStruct((L, 8, 2, n_bgh, 8, 128), jnp.float32),
    )(e2)


@jax.jit
def kernel(x, table, W_down, W_up):
    B, L = x.shape
    vocab = table.shape[0]
    n = B * L

    idx = x.T.reshape(n).astype(jnp.int32)

    tr_sh = TR_COLS.bit_length() - 1
    blk = idx >> tr_sh
    off = idx & (TR_COLS - 1)
    phys_idx = (blk << tr_sh) + ((off & (TR_COLS // 2 - 1)) << 1) \
        + (off >> (tr_sh - 1))

    g = _tc_transform(table.T, W_down, W_up, vocab)
    g_rows = g.reshape(g.shape[0] * g.shape[1] * 2, DIM)
    e2 = _sc_gather(g_rows, phys_idx, n, B)
    y = _tc_format(e2, B, L)
    return y.transpose(2, 3, 5, 0, 1, 4).reshape(B, L, DIM)


# speedup vs baseline: 2.2515x; 1.0275x over previous
import functools

import jax
import jax.numpy as jnp
from jax import lax
from jax.experimental import pallas as pl
from jax.experimental.pallas import tpu as pltpu
from jax.experimental.pallas import tpu_sc as plsc

DIM = 64
NC = 2
NS = 16
NW = NC * NS
LANES = 16

GATHER_CHUNK = 1024
TR_COLS = 32768
FMT_ROWS = 8192


def _tc_transform(tableT, W_down, W_up, vocab):
    def tr_kernel(tt_ref, wd_ref, wu_ref, out_ref):
        wc = jnp.dot(wu_ref[...], wd_ref[...],
                     preferred_element_type=jnp.float32)
        z = lax.dot_general(tt_ref[...], wc, (((0,), (1,)), ((), ())),
                            preferred_element_type=jnp.float32)
        out_ref[0, :, 0:DIM] = z[: TR_COLS // 2]
        out_ref[0, :, DIM:128] = z[TR_COLS // 2 :]

    grid = (vocab + TR_COLS - 1) // TR_COLS
    return pl.pallas_call(
        tr_kernel,
        grid=(grid,),
        in_specs=[
            pl.BlockSpec((DIM, TR_COLS), lambda i: (0, i)),
            pl.BlockSpec((DIM // 2, DIM), lambda i: (0, 0)),
            pl.BlockSpec((DIM, DIM // 2), lambda i: (0, 0)),
        ],
        out_specs=pl.BlockSpec((1, TR_COLS // 2, 128), lambda i: (i, 0, 0)),
        out_shape=jax.ShapeDtypeStruct((grid, TR_COLS // 2, 128), jnp.float32),
        compiler_params=pltpu.CompilerParams(
            fuse_transposed_lhs_in_matmul=True,
            vmem_limit_bytes=56 * 1024 * 1024,
        ),
    )(tableT, W_down, W_up)


def _sc_gather(src, flat_idx, n, B):
    half = B // 2
    pairs = n // 2
    per_w = pairs // NW
    chunk = min(GATHER_CHUNK // 2, per_w)
    n_chunks = per_w // chunk
    mesh = plsc.VectorSubcoreMesh(
        core_axis_name="c", subcore_axis_name="s",
        num_cores=NC, num_subcores=NS)

    @functools.partial(
        pl.kernel,
        out_type=jax.ShapeDtypeStruct((pairs, 2 * DIM), jnp.float32),
        mesh=mesh,
        scratch_types=[
            pltpu.VMEM((chunk,), jnp.int32),
            pltpu.VMEM((chunk,), jnp.int32),
            pltpu.VMEM((chunk, DIM), jnp.float32),
            pltpu.VMEM((chunk, DIM), jnp.float32),
            pltpu.SemaphoreType.DMA,
            pltpu.SemaphoreType.DMA,
        ],
        compiler_params=pltpu.CompilerParams(use_tc_tiling_on_sc=False),
    )
    def gather_kernel(src_hbm, idx_hbm, out_hbm,
                      idx_a, idx_b, rows_a, rows_b, sem_a, sem_b):
        wid = lax.axis_index("s") * NC + lax.axis_index("c")
        base = wid * per_w

        def body(i, carry):
            m0 = pl.multiple_of(base + i * chunk, chunk)
            l = m0 >> 13
            u0 = m0 - (l << 13)
            pos = pl.multiple_of((l << 14) + u0, chunk)
            pltpu.sync_copy(idx_hbm.at[pl.ds(pos, chunk)], idx_a)
            pltpu.sync_copy(idx_hbm.at[pl.ds(pos + half, chunk)], idx_b)
            ca = pltpu.async_copy(src_hbm.at[idx_a], rows_a, sem_a)
            cb = pltpu.async_copy(src_hbm.at[idx_b], rows_b, sem_b)
            ca.wait()
            pltpu.sync_copy(rows_a, out_hbm.at[pl.ds(m0, chunk), pl.ds(0, DIM)])
            cb.wait()
            pltpu.sync_copy(rows_b, out_hbm.at[pl.ds(m0, chunk), pl.ds(DIM, DIM)])
            return carry

        lax.fori_loop(0, n_chunks, body, 0)

    return gather_kernel(src, flat_idx)


def _tc_format(e2, B, L):
    sub = FMT_ROWS // 128

    def fmt_kernel(e_ref, out_ref):
        z = jnp.transpose(e_ref[...])
        for s in range(sub):
            zs = z[:, s * 128 : (s + 1) * 128]
            out_ref[0, :, 0, s, :, :] = zs[0:DIM].reshape(8, 8, 128)
            out_ref[0, :, 1, s, :, :] = zs[DIM:128].reshape(8, 8, 128)

    n_bgh = B // 256
    grid = (L, n_bgh // sub)
    return pl.pallas_call(
        fmt_kernel,
        grid=grid,
        in_specs=[
            pl.BlockSpec((FMT_ROWS, 128),
                         lambda l, k: (l * (n_bgh // sub) + k, 0)),
        ],
        out_specs=pl.BlockSpec((1, 8, 2, sub, 8, 128),
                               lambda l, k: (l, 0, 0, k, 0, 0)),
        out_shape=jax.ShapeDtypeStruct((L, 8, 2, n_bgh, 8, 128), jnp.float32),
    )(e2)


@jax.jit
def kernel(x, table, W_down, W_up):
    B, L = x.shape
    vocab = table.shape[0]
    n = B * L

    idx = x.T.reshape(n).astype(jnp.int32)

    tr_sh = TR_COLS.bit_length() - 1
    blk = idx >> tr_sh
    off = idx & (TR_COLS - 1)
    phys_idx = (blk << tr_sh) + ((off & (TR_COLS // 2 - 1)) << 1) \
        + (off >> (tr_sh - 1))

    g = _tc_transform(table.T, W_down, W_up, vocab)
    g_rows = g.reshape(g.shape[0] * g.shape[1] * 2, DIM)
    e2 = _sc_gather(g_rows, phys_idx, n, B)
    y = _tc_format(e2, B, L)
    return y.transpose(2, 3, 5, 0, 1, 4).reshape(B, L, DIM)


# speedup vs baseline: 2.3905x; 1.0617x over previous
import functools

import jax
import jax.numpy as jnp
from jax import lax
from jax.experimental import pallas as pl
from jax.experimental.pallas import tpu as pltpu
from jax.experimental.pallas import tpu_sc as plsc

DIM = 64
NC = 2
NS = 16
NW = NC * NS
LANES = 16

GATHER_CHUNK = 1024
TR_COLS = 32768
FMT_ROWS = 8192


def _tc_transform(tableT, W_down, W_up, vocab):
    def tr_kernel(tt_ref, wd_ref, wu_ref, out_ref):
        wc = jnp.dot(wu_ref[...], wd_ref[...],
                     preferred_element_type=jnp.float32)
        z = lax.dot_general(tt_ref[...], wc, (((0,), (1,)), ((), ())),
                            preferred_element_type=jnp.float32)
        out_ref[0, :, 0:DIM] = z[: TR_COLS // 2]
        out_ref[0, :, DIM:128] = z[TR_COLS // 2 :]

    grid = (vocab + TR_COLS - 1) // TR_COLS
    return pl.pallas_call(
        tr_kernel,
        grid=(grid,),
        in_specs=[
            pl.BlockSpec((DIM, TR_COLS), lambda i: (0, i)),
            pl.BlockSpec((DIM // 2, DIM), lambda i: (0, 0)),
            pl.BlockSpec((DIM, DIM // 2), lambda i: (0, 0)),
        ],
        out_specs=pl.BlockSpec((1, TR_COLS // 2, 128), lambda i: (i, 0, 0)),
        out_shape=jax.ShapeDtypeStruct((grid, TR_COLS // 2, 128), jnp.float32),
        compiler_params=pltpu.CompilerParams(
            fuse_transposed_lhs_in_matmul=True,
            vmem_limit_bytes=56 * 1024 * 1024,
        ),
    )(tableT, W_down, W_up)


def _sc_gather(src, flat_idx, n, B):
    half = B // 2
    pairs = n // 2
    per_w = pairs // NW
    chunk = min(GATHER_CHUNK // 2, per_w)
    n_chunks = per_w // chunk
    mesh = plsc.VectorSubcoreMesh(
        core_axis_name="c", subcore_axis_name="s",
        num_cores=NC, num_subcores=NS)

    @functools.partial(
        pl.kernel,
        out_type=jax.ShapeDtypeStruct((pairs, 2 * DIM), jnp.float32),
        mesh=mesh,
        scratch_types=[
            pltpu.VMEM((chunk,), jnp.int32),
            pltpu.VMEM((chunk,), jnp.int32),
            pltpu.VMEM((chunk, DIM), jnp.float32),
            pltpu.VMEM((chunk, DIM), jnp.float32),
            pltpu.SemaphoreType.DMA,
            pltpu.SemaphoreType.DMA,
            pltpu.SemaphoreType.DMA,
            pltpu.SemaphoreType.DMA,
        ],
        compiler_params=pltpu.CompilerParams(use_tc_tiling_on_sc=False),
    )
    def gather_kernel(src_hbm, idx_hbm, out_hbm,
                      idx_a, idx_b, rows_a, rows_b,
                      sem_a, sem_b, osem_a, osem_b):
        wid = lax.axis_index("s") * NC + lax.axis_index("c")
        base = wid * per_w

        def out_slot(m0, lane0):
            return out_hbm.at[pl.ds(m0, chunk), pl.ds(lane0, DIM)]

        def body(i, carry):
            m0 = pl.multiple_of(base + i * chunk, chunk)
            l = m0 >> 13
            u0 = m0 - (l << 13)
            pos = pl.multiple_of((l << 14) + u0, chunk)
            pltpu.sync_copy(idx_hbm.at[pl.ds(pos, chunk)], idx_a)
            pltpu.sync_copy(idx_hbm.at[pl.ds(pos + half, chunk)], idx_b)

            @pl.when(i > 0)
            def _():
                pltpu.make_async_copy(rows_a, out_slot(m0, 0), osem_a).wait()
                pltpu.make_async_copy(rows_b, out_slot(m0, DIM), osem_b).wait()

            ca = pltpu.async_copy(src_hbm.at[idx_a], rows_a, sem_a)
            cb = pltpu.async_copy(src_hbm.at[idx_b], rows_b, sem_b)
            ca.wait()
            pltpu.async_copy(rows_a, out_slot(m0, 0), osem_a)
            cb.wait()
            pltpu.async_copy(rows_b, out_slot(m0, DIM), osem_b)
            return carry

        lax.fori_loop(0, n_chunks, body, 0)
        last = pl.multiple_of(base + (n_chunks - 1) * chunk, chunk)
        pltpu.make_async_copy(rows_a, out_slot(last, 0), osem_a).wait()
        pltpu.make_async_copy(rows_b, out_slot(last, DIM), osem_b).wait()

    return gather_kernel(src, flat_idx)


def _tc_format(e2, B, L):
    sub = FMT_ROWS // 128

    def fmt_kernel(e_ref, out_ref):
        z = jnp.transpose(e_ref[...])
        for s in range(sub):
            zs = z[:, s * 128 : (s + 1) * 128]
            out_ref[0, :, 0, s, :, :] = zs[0:DIM].reshape(8, 8, 128)
            out_ref[0, :, 1, s, :, :] = zs[DIM:128].reshape(8, 8, 128)

    n_bgh = B // 256
    grid = (L, n_bgh // sub)
    return pl.pallas_call(
        fmt_kernel,
        grid=grid,
        in_specs=[
            pl.BlockSpec((FMT_ROWS, 128),
                         lambda l, k: (l * (n_bgh // sub) + k, 0)),
        ],
        out_specs=pl.BlockSpec((1, 8, 2, sub, 8, 128),
                               lambda l, k: (l, 0, 0, k, 0, 0)),
        out_shape=jax.ShapeDtypeStruct((L, 8, 2, n_bgh, 8, 128), jnp.float32),
    )(e2)


@jax.jit
def kernel(x, table, W_down, W_up):
    B, L = x.shape
    vocab = table.shape[0]
    n = B * L

    idx = x.T.reshape(n).astype(jnp.int32)

    tr_sh = TR_COLS.bit_length() - 1
    blk = idx >> tr_sh
    off = idx & (TR_COLS - 1)
    phys_idx = (blk << tr_sh) + ((off & (TR_COLS // 2 - 1)) << 1) \
        + (off >> (tr_sh - 1))

    g = _tc_transform(table.T, W_down, W_up, vocab)
    g_rows = g.reshape(g.shape[0] * g.shape[1] * 2, DIM)
    e2 = _sc_gather(g_rows, phys_idx, n, B)
    y = _tc_format(e2, B, L)
    return y.transpose(2, 3, 5, 0, 1, 4).reshape(B, L, DIM)
